# baseline (speedup 1.0000x reference)
import functools

import jax
import jax.numpy as jnp
from jax import lax
from jax.experimental import pallas as pl
from jax.experimental.pallas import tpu as pltpu
from jax.experimental.pallas import tpu_sc as plsc

N_NODES_PAD = 10240
EDGES = 320000
NC, NS = 2, 16
PER_SUBCORE = EDGES // NS
NSTREAM = 10
STREAM = PER_SUBCORE // NSTREAM
SLICE = N_NODES_PAD // NS

_mesh = plsc.VectorSubcoreMesh(
    core_axis_name="c", subcore_axis_name="s", num_cores=NC, num_subcores=NS
)


@functools.partial(
    pl.kernel,
    out_type=jax.ShapeDtypeStruct((NC, N_NODES_PAD), jnp.int32),
    mesh=_mesh,
    scratch_types=[
        [pltpu.VMEM((STREAM,), jnp.int32) for _ in range(NSTREAM)],
        pltpu.VMEM((STREAM,), jnp.int32),
        pltpu.VMEM((SLICE,), jnp.int32),
        pltpu.VMEM_SHARED((N_NODES_PAD,), jnp.int32),
        pltpu.SemaphoreType.DMA,
        pltpu.SemaphoreType.DMA,
    ],
    compiler_params=pltpu.CompilerParams(use_tc_tiling_on_sc=False),
)
def _degree_sc(edge_hbm, deg_hbm, idx_v, ones_v, zero_v, hist_s, sem, sem2):
    c = lax.axis_index("c")
    s = lax.axis_index("s")

    idx_cps = [
        pltpu.async_copy(
            edge_hbm.at[c, pl.ds(s * PER_SUBCORE + k * STREAM, STREAM)],
            idx_v[k],
            sem,
        )
        for k in range(NSTREAM)
    ]

    def fill_ones(i, carry):
        ones_v[pl.ds(pl.multiple_of(i * 16, 16), 16)] = jnp.full((16,), 1, jnp.int32)
        return carry

    lax.fori_loop(0, STREAM // 16, fill_ones, 0)
    for k in range(SLICE // 16):
        zero_v[pl.ds(k * 16, 16)] = jnp.zeros((16,), jnp.int32)

    pltpu.sync_copy(zero_v, hist_s.at[pl.ds(s * SLICE, SLICE)])
    for cp in idx_cps:
        cp.wait()
    plsc.subcore_barrier()

    adds = [
        pltpu.async_copy(ones_v, hist_s.at[idx_v[k]], sem2, add=True)
        for k in range(NSTREAM)
    ]
    for cp in adds:
        cp.wait()
    plsc.subcore_barrier()

    pltpu.sync_copy(hist_s.at[pl.ds(s * SLICE, SLICE)],
                    deg_hbm.at[c, pl.ds(s * SLICE, SLICE)])


def kernel(x, edge_index):
    ei = edge_index.astype(jnp.int32)
    deg = _degree_sc(ei)
    out_dtype = jax.dtypes.canonicalize_dtype(jnp.int64)
    out_degree = deg[0, :10000].astype(out_dtype)
    in_degree = deg[1, :10000].astype(out_dtype)
    return x, in_degree, out_degree


# speedup vs baseline: 1.0249x; 1.0249x over previous
import functools

import jax
import jax.numpy as jnp
from jax import lax
from jax.experimental import pallas as pl
from jax.experimental.pallas import tpu as pltpu
from jax.experimental.pallas import tpu_sc as plsc

N_NODES_PAD = 10240
EDGES = 320000
NC, NS = 2, 16
PER_SUBCORE = EDGES // NS
NSTREAM = 5
STREAM = PER_SUBCORE // NSTREAM
SLICE = N_NODES_PAD // NS

_mesh = plsc.VectorSubcoreMesh(
    core_axis_name="c", subcore_axis_name="s", num_cores=NC, num_subcores=NS
)


@functools.partial(
    pl.kernel,
    out_type=jax.ShapeDtypeStruct((NC * N_NODES_PAD,), jnp.int32),
    mesh=_mesh,
    scratch_types=[
        [pltpu.VMEM((STREAM,), jnp.int32) for _ in range(NSTREAM)],
        pltpu.VMEM((STREAM,), jnp.int32),
        pltpu.VMEM((SLICE,), jnp.int32),
        pltpu.VMEM_SHARED((N_NODES_PAD,), jnp.int32),
        pltpu.SemaphoreType.DMA,
        pltpu.SemaphoreType.DMA,
    ],
    compiler_params=pltpu.CompilerParams(use_tc_tiling_on_sc=False),
)
def _degree_sc(edge_hbm, deg_hbm, idx_v, ones_v, zero_v, hist_s, sem, sem2):
    c = lax.axis_index("c")
    s = lax.axis_index("s")

    idx_cps = [
        pltpu.async_copy(
            edge_hbm.at[c, pl.ds(s * PER_SUBCORE + k * STREAM, STREAM)],
            idx_v[k],
            sem,
        )
        for k in range(NSTREAM)
    ]

    def fill_ones(i, carry):
        ones_v[pl.ds(pl.multiple_of(i * 16, 16), 16)] = jnp.full((16,), 1, jnp.int32)
        return carry

    lax.fori_loop(0, STREAM // 16, fill_ones, 0)
    for k in range(SLICE // 16):
        zero_v[pl.ds(k * 16, 16)] = jnp.zeros((16,), jnp.int32)

    pltpu.sync_copy(zero_v, hist_s.at[pl.ds(s * SLICE, SLICE)])
    plsc.subcore_barrier()

    adds = []
    for k in range(NSTREAM):
        idx_cps[k].wait()
        adds.append(
            pltpu.async_copy(ones_v, hist_s.at[idx_v[k]], sem2, add=True)
        )
    for cp in adds:
        cp.wait()
    plsc.subcore_barrier()

    off = pl.multiple_of(c * N_NODES_PAD + s * SLICE, 8)
    pltpu.sync_copy(hist_s.at[pl.ds(s * SLICE, SLICE)],
                    deg_hbm.at[pl.ds(off, SLICE)])


def kernel(x, edge_index):
    ei = edge_index.astype(jnp.int32)
    deg = _degree_sc(ei)
    out_dtype = jax.dtypes.canonicalize_dtype(jnp.int64)
    out_degree = deg[:10000].astype(out_dtype)
    in_degree = deg[N_NODES_PAD:N_NODES_PAD + 10000].astype(out_dtype)
    return x, in_degree, out_degree


# speedup vs baseline: 1.0697x; 1.0437x over previous
import functools

import jax
import jax.numpy as jnp
from jax import lax
from jax.experimental import pallas as pl
from jax.experimental.pallas import tpu as pltpu
from jax.experimental.pallas import tpu_sc as plsc

N_NODES_PAD = 10240
EDGES = 320000
NC, NS = 2, 16
PER_SUBCORE = EDGES // NS
NSTREAM = 5
STREAM = PER_SUBCORE // NSTREAM
SLICE = N_NODES_PAD // NS

_mesh = plsc.VectorSubcoreMesh(
    core_axis_name="c", subcore_axis_name="s", num_cores=NC, num_subcores=NS
)


@functools.partial(
    pl.kernel,
    out_type=(
        jax.ShapeDtypeStruct((10000,), jnp.int32),
        jax.ShapeDtypeStruct((10000,), jnp.int32),
    ),
    mesh=_mesh,
    scratch_types=[
        [pltpu.VMEM((STREAM,), jnp.int32) for _ in range(NSTREAM)],
        pltpu.VMEM((STREAM,), jnp.int32),
        pltpu.VMEM((SLICE,), jnp.int32),
        pltpu.VMEM_SHARED((N_NODES_PAD,), jnp.int32),
        pltpu.SemaphoreType.DMA,
        pltpu.SemaphoreType.DMA,
    ],
    compiler_params=pltpu.CompilerParams(use_tc_tiling_on_sc=False),
)
def _degree_sc(edge_hbm, out0_hbm, out1_hbm, idx_v, ones_v, zero_v, hist_s, sem, sem2):
    c = lax.axis_index("c")
    s = lax.axis_index("s")

    idx_cps = [
        pltpu.async_copy(
            edge_hbm.at[c, pl.ds(s * PER_SUBCORE + k * STREAM, STREAM)],
            idx_v[k],
            sem,
        )
        for k in range(NSTREAM)
    ]

    def fill_ones(i, carry):
        ones_v[pl.ds(pl.multiple_of(i * 16, 16), 16)] = jnp.full((16,), 1, jnp.int32)
        return carry

    lax.fori_loop(0, STREAM // 16, fill_ones, 0)
    for k in range(SLICE // 16):
        zero_v[pl.ds(k * 16, 16)] = jnp.zeros((16,), jnp.int32)

    pltpu.sync_copy(zero_v, hist_s.at[pl.ds(s * SLICE, SLICE)])
    plsc.subcore_barrier()

    adds = []
    for k in range(NSTREAM):
        idx_cps[k].wait()
        adds.append(
            pltpu.async_copy(ones_v, hist_s.at[idx_v[k]], sem2, add=True)
        )
    for cp in adds:
        cp.wait()
    plsc.subcore_barrier()

    @pl.when(s < NS - 1)
    def _full_slice():
        for dst in (out0_hbm, out1_hbm):
            @pl.when((c == 0) == (dst is out0_hbm))
            def _():
                pltpu.sync_copy(hist_s.at[pl.ds(s * SLICE, SLICE)],
                                dst.at[pl.ds(s * SLICE, SLICE)])

    @pl.when(s == NS - 1)
    def _tail_slice():
        tail = 10000 - (NS - 1) * SLICE
        for dst in (out0_hbm, out1_hbm):
            @pl.when((c == 0) == (dst is out0_hbm))
            def _():
                pltpu.sync_copy(hist_s.at[pl.ds((NS - 1) * SLICE, tail)],
                                dst.at[pl.ds((NS - 1) * SLICE, tail)])


def kernel(x, edge_index):
    ei = edge_index.astype(jnp.int32)
    out_degree, in_degree = _degree_sc(ei)
    out_dtype = jax.dtypes.canonicalize_dtype(jnp.int64)
    return x, in_degree.astype(out_dtype), out_degree.astype(out_dtype)


# speedup vs baseline: 1.1372x; 1.0631x over previous
import functools

import jax
import jax.numpy as jnp
from jax import lax
from jax.experimental import pallas as pl
from jax.experimental.pallas import tpu as pltpu
from jax.experimental.pallas import tpu_sc as plsc

N_NODES_PAD = 10240
EDGES = 320000
NC, NS = 2, 16
PER_SUBCORE = EDGES // NS
NSTREAM = 5
STREAM = PER_SUBCORE // NSTREAM
SLICE = N_NODES_PAD // NS

_mesh = plsc.VectorSubcoreMesh(
    core_axis_name="c", subcore_axis_name="s", num_cores=NC, num_subcores=NS
)


@functools.partial(
    pl.kernel,
    out_type=(
        jax.ShapeDtypeStruct((10000,), jnp.int32),
        jax.ShapeDtypeStruct((10000,), jnp.int32),
    ),
    mesh=_mesh,
    scratch_types=[
        [pltpu.VMEM((STREAM,), jnp.int32) for _ in range(NSTREAM)],
        pltpu.VMEM((STREAM,), jnp.int32),
        pltpu.VMEM((SLICE,), jnp.int32),
        pltpu.VMEM_SHARED((N_NODES_PAD,), jnp.int32),
        pltpu.SemaphoreType.DMA,
        pltpu.SemaphoreType.DMA,
    ],
    compiler_params=pltpu.CompilerParams(use_tc_tiling_on_sc=False),
)
def _degree_sc(edge_hbm, out0_hbm, out1_hbm, idx_v, ones_v, zero_v, hist_s, sem, sem2):
    c = lax.axis_index("c")
    s = lax.axis_index("s")

    idx_cps = [
        pltpu.async_copy(
            edge_hbm.at[c, pl.ds(s * PER_SUBCORE + k * STREAM, STREAM)],
            idx_v[k],
            sem,
        )
        for k in range(NSTREAM)
    ]

    def fill_ones(i, carry):
        ones_v[pl.ds(pl.multiple_of(i * 16, 16), 16)] = jnp.full((16,), 1, jnp.int32)
        return carry

    lax.fori_loop(0, STREAM // 16, fill_ones, 0)
    for k in range(SLICE // 16):
        zero_v[pl.ds(k * 16, 16)] = jnp.zeros((16,), jnp.int32)

    pltpu.sync_copy(zero_v, hist_s.at[pl.ds(s * SLICE, SLICE)])
    plsc.subcore_barrier()

    adds = []
    for k in range(NSTREAM):
        idx_cps[k].wait()
        adds.append(
            pltpu.async_copy(ones_v, hist_s.at[idx_v[k]], sem2, add=True)
        )
    for cp in adds:
        cp.wait()
    plsc.subcore_barrier()

    @pl.when(s < NS - 1)
    def _full_slice():
        for dst in (out0_hbm, out1_hbm):
            @pl.when((c == 0) == (dst is out0_hbm))
            def _():
                pltpu.sync_copy(hist_s.at[pl.ds(s * SLICE, SLICE)],
                                dst.at[pl.ds(s * SLICE, SLICE)])

    @pl.when(s == NS - 1)
    def _tail_slice():
        tail = 10000 - (NS - 1) * SLICE
        for dst in (out0_hbm, out1_hbm):
            @pl.when((c == 0) == (dst is out0_hbm))
            def _():
                pltpu.sync_copy(hist_s.at[pl.ds((NS - 1) * SLICE, tail)],
                                dst.at[pl.ds((NS - 1) * SLICE, tail)])


def _copy_body(x_ref, o_ref):
    o_ref[...] = x_ref[...]


_copy_x = pl.pallas_call(
    _copy_body,
    grid=(5,),
    in_specs=[pl.BlockSpec((2000, 128), lambda i: (i, 0))],
    out_specs=pl.BlockSpec((2000, 128), lambda i: (i, 0)),
    out_shape=jax.ShapeDtypeStruct((10000, 128), jnp.float32),
)


def kernel(x, edge_index):
    ei = edge_index.astype(jnp.int32)
    out_degree, in_degree = _degree_sc(ei)
    x_out = _copy_x(x)
    out_dtype = jax.dtypes.canonicalize_dtype(jnp.int64)
    return x_out, in_degree.astype(out_dtype), out_degree.astype(out_dtype)
